# baseline (speedup 1.0000x reference)
import functools

import jax
import jax.numpy as jnp
from jax import lax
from jax.experimental import pallas as pl
from jax.experimental.pallas import tpu as pltpu
from jax.experimental.pallas import tpu_sc as plsc

B = 32768
D = 512
NW = 32
BPW = B // NW
CH = 64
NCH = BPW // CH
NB = 3
L = 16
U = 8


@functools.partial(
    pl.kernel,
    mesh=plsc.VectorSubcoreMesh(core_axis_name="c", subcore_axis_name="s"),
    out_type=jax.ShapeDtypeStruct((B * D,), jnp.float32),
    compiler_params=pltpu.CompilerParams(needs_layout_passes=False),
    scratch_types=[
        pltpu.VMEM((NCH, CH), jnp.int32),
        pltpu.VMEM((NB * CH * D,), jnp.float32),
        pltpu.VMEM((3 * D,), jnp.float32),
        pltpu.SemaphoreType.DMA,
        pltpu.SemaphoreType.DMA,
        pltpu.SemaphoreType.DMA,
    ],
)
def _emb(x_hbm, table_hbm, out_hbm, idx_v, buf, tab_v, s0, s1, s2):
    ssems = (s0, s1, s2)
    wid = lax.axis_index("s") * 2 + lax.axis_index("c")
    base = wid * BPW

    pltpu.sync_copy(table_hbm, tab_v)
    pltpu.sync_copy(x_hbm.at[wid], idx_v)

    sd = [None] * NB
    for c in range(NCH):
        b = c % NB
        if sd[b] is not None:
            sd[b].wait()
        for q in range(CH // L):
            iv = idx_v[c, pl.ds(q * L, L)]
            src0 = iv * D
            dst0 = (b * CH + q * L) * D + lax.iota(jnp.int32, L) * D

            def body(i, cols, src0=src0, dst0=dst0):
                for u in range(U):
                    val = plsc.load_gather(tab_v, [src0 + cols])
                    plsc.store_scatter(buf, [dst0 + cols], val)
                    cols = cols + 1
                return cols

            lax.fori_loop(0, D // U, body, jnp.zeros((L,), jnp.int32))
        sd[b] = pltpu.async_copy(
            buf.at[pl.ds(b * CH * D, CH * D)],
            out_hbm.at[pl.ds((base + c * CH) * D, CH * D)], ssems[b])
    for b in range(NB):
        sd[b].wait()


def kernel(x, table):
    xw = x.reshape(NW, NCH, CH).astype(jnp.int32)
    out = _emb(xw, table.reshape(-1).astype(jnp.float32))
    return out.reshape(x.shape + (table.shape[1],))


# speedup vs baseline: 7.4525x; 7.4525x over previous
import functools

import jax
import jax.numpy as jnp
from jax import lax
from jax.experimental import pallas as pl
from jax.experimental.pallas import tpu as pltpu
from jax.experimental.pallas import tpu_sc as plsc

B = 32768
D = 512
NW = 32
BPW = B // NW
CH = 64
NCH = BPW // CH
NB = 3
L = 16
QR = 328
NG = BPW // (4 * L)


@functools.partial(
    pl.kernel,
    mesh=plsc.VectorSubcoreMesh(core_axis_name="c", subcore_axis_name="s"),
    out_type=jax.ShapeDtypeStruct((B, D), jnp.float32),
    compiler_params=pltpu.CompilerParams(needs_layout_passes=False),
    scratch_types=[
        pltpu.VMEM((8, 128), jnp.int32),
        pltpu.VMEM((NCH, CH), jnp.int32),
        pltpu.VMEM((NB, CH, D), jnp.float32),
        pltpu.VMEM((3 * D,), jnp.float32),
        pltpu.VMEM((8, D), jnp.float32),
        pltpu.HBM((2 * QR, D), jnp.float32),
        pltpu.SemaphoreType.DMA,
        pltpu.SemaphoreType.DMA,
        pltpu.SemaphoreType.DMA,
        pltpu.SemaphoreType.DMA,
        pltpu.SemaphoreType.DMA,
        pltpu.SemaphoreType.DMA,
    ],
)
def _emb(x_hbm, table_hbm, out_hbm, idx_v, gidx, buf, tab_v, stage_v,
         quadtab, g0, g1, g2, s0, s1, s2):
    gsems = (g0, g1, g2)
    ssems = (s0, s1, s2)
    core = lax.axis_index("c")
    sid = lax.axis_index("s")
    wid = sid * 2 + core
    base = wid * BPW
    iota = lax.iota(jnp.int32, L)

    pltpu.sync_copy(table_hbm, tab_v)
    pltpu.sync_copy(x_hbm.at[wid], idx_v)

    for m in range(3):
        gnum = sid + 16 * m

        @pl.when(gnum < 41)
        def _build(gnum=gnum):
            for k in range(8):
                q = 2 * gnum + (k >> 2)
                j = k & 3
                p = (27, 9, 3, 1)[j]
                digit = lax.rem(lax.div(q, p), 3)
                srow = digit * D
                for kk in range(D // L):
                    val = plsc.load_gather(tab_v, [srow + kk * L + iota])
                    stage_v[k, pl.ds(kk * L, L)] = val
            pltpu.sync_copy(
                stage_v,
                quadtab.at[pl.ds(pl.multiple_of(QR * core + 8 * gnum, 8), 8)])

    cbase = core * QR
    for G in range(NG):
        pos = 64 * G + 4 * iota
        xs = []
        for j in range(4):
            rows = (pos + j) // 128
            cols = (pos + j) % 128
            xs.append(plsc.load_gather(idx_v, [rows, cols]))
        qv = ((xs[0] * 3 + xs[1]) * 3 + xs[2]) * 3 + xs[3]
        gbase = cbase + qv * 4
        for j in range(4):
            rows2 = (pos + j) // CH
            cols2 = (pos + j) % CH
            plsc.store_scatter(gidx, [rows2, cols2], gbase + j)

    plsc.subcore_barrier()

    gd = [None] * NB
    sd = [None] * NB
    for b in range(NB):
        gd[b] = pltpu.async_copy(quadtab.at[gidx.at[b]], buf.at[b], gsems[b])
    for c in range(NCH):
        b = c % NB
        gd[b].wait()
        sd[b] = pltpu.async_copy(
            buf.at[b], out_hbm.at[pl.ds(base + c * CH, CH)], ssems[b])
        n = c - 1 + NB
        if c >= 1 and n < NCH:
            bm = (c - 1) % NB
            sd[bm].wait()
            gd[bm] = pltpu.async_copy(
                quadtab.at[gidx.at[n]], buf.at[bm], gsems[bm])
    for c in range(NCH - NB, NCH):
        sd[c % NB].wait()


def kernel(x, table):
    xw = x.reshape(NW, 8, 128).astype(jnp.int32)
    out = _emb(xw, table.reshape(-1).astype(jnp.float32))
    return out.reshape(x.shape + (table.shape[1],))
